# Initial kernel scaffold; baseline (speedup 1.0000x reference)
import jax
import jax.numpy as jnp
from jax.experimental import pallas as pl


def kernel(q, c, r, que_table, concept_emb, W, b):
    raise NotImplementedError("write your pallas kernel here")


# speedup vs baseline: 3.4215x; 3.4215x over previous
import functools

import jax
import jax.numpy as jnp
from jax import lax
from jax.experimental import pallas as pl
from jax.experimental.pallas import tpu as pltpu
from jax.experimental.pallas import tpu_sc as plsc

NUM_Q = 100000
NUM_C = 1000
EMB = 128
B = 1024
L = 200
N = B * L
NC = 2
NS = 16
NW = NC * NS
TOK_PER_W = N // NW
T = 128
CHUNKS = TOK_PER_W // T


def _sc_body(q_hbm, c_hbm, que_hbm, conc_hbm, embq_hbm, embc_hbm,
             qidx, cidx, rows_q, rows_c, embc_v, sem):
    wid = lax.axis_index("s") * NC + lax.axis_index("c")

    def chunk_body(i, carry):
        base = wid * TOK_PER_W + i * T
        pltpu.sync_copy(q_hbm.at[pl.ds(base, T)], qidx)
        pltpu.sync_copy(c_hbm.at[pl.ds(base * 4, T * 4)], cidx)
        pltpu.async_copy(que_hbm.at[qidx], rows_q, sem).wait()
        for j in range(4):
            pltpu.async_copy(conc_hbm.at[cidx.at[pl.ds(j * 128, 128)]],
                             rows_c.at[pl.ds(j * 128, 128)], sem).wait()

        def red_body(t, c2):
            for h in range(EMB // 16):
                s = pl.ds(h * 16, 16)
                v = (rows_c[4 * t, s] + rows_c[4 * t + 1, s]
                     + rows_c[4 * t + 2, s] + rows_c[4 * t + 3, s]) * 0.25
                embc_v[t, s] = v
            return c2
        lax.fori_loop(0, T, red_body, 0)

        pltpu.sync_copy(rows_q, embq_hbm.at[pl.ds(base, T)])
        pltpu.sync_copy(embc_v, embc_hbm.at[pl.ds(base, T)])
        return carry

    lax.fori_loop(0, CHUNKS, chunk_body, 0)


@jax.jit
def _sc_gather(q_flat, c_rows, que_table, concept_emb):
    mesh = plsc.VectorSubcoreMesh(core_axis_name="c", subcore_axis_name="s")
    fn = pl.kernel(
        _sc_body,
        out_type=[jax.ShapeDtypeStruct((N, EMB), jnp.float32),
                  jax.ShapeDtypeStruct((N, EMB), jnp.float32)],
        mesh=mesh,
        scratch_types=[
            pltpu.VMEM((T,), jnp.int32),
            pltpu.VMEM((4 * T,), jnp.int32),
            pltpu.VMEM((T, EMB), jnp.float32),
            pltpu.VMEM((4 * T, EMB), jnp.float32),
            pltpu.VMEM((T, EMB), jnp.float32),
            pltpu.SemaphoreType.DMA,
        ],
    )
    return fn(q_flat, c_rows, que_table, concept_emb)


TB = 512


def _tc_body(q_ref, c_ref, r_ref, w_ref, b_ref, qc_ref, x_ref, qca_ref):
    qc = jnp.concatenate([q_ref[...], c_ref[...]], axis=1)
    qc_ref[...] = qc
    x_ref[...] = jnp.dot(qc, w_ref[...],
                         preferred_element_type=jnp.float32) + b_ref[...]
    rf = r_ref[...]
    qca_ref[...] = jnp.concatenate([qc * (1.0 - rf), qc * rf], axis=1)


@jax.jit
def _tc_merge(emb_q, emb_c, r1, W, b2):
    grid = (N // TB,)
    return pl.pallas_call(
        _tc_body,
        grid=grid,
        in_specs=[
            pl.BlockSpec((TB, EMB), lambda i: (i, 0)),
            pl.BlockSpec((TB, EMB), lambda i: (i, 0)),
            pl.BlockSpec((TB, 1), lambda i: (i, 0)),
            pl.BlockSpec((2 * EMB, EMB), lambda i: (0, 0)),
            pl.BlockSpec((1, EMB), lambda i: (0, 0)),
        ],
        out_specs=[
            pl.BlockSpec((TB, 2 * EMB), lambda i: (i, 0)),
            pl.BlockSpec((TB, EMB), lambda i: (i, 0)),
            pl.BlockSpec((TB, 4 * EMB), lambda i: (i, 0)),
        ],
        out_shape=[
            jax.ShapeDtypeStruct((N, 2 * EMB), jnp.float32),
            jax.ShapeDtypeStruct((N, EMB), jnp.float32),
            jax.ShapeDtypeStruct((N, 4 * EMB), jnp.float32),
        ],
    )(emb_q, emb_c, r1, W, b2)


def kernel(q, c, r, que_table, concept_emb, W, b):
    q_flat = q.reshape(-1).astype(jnp.int32)
    c_rows = c.astype(jnp.int32).reshape(N * 4)
    r1 = r.astype(jnp.float32).reshape(N, 1)
    emb_q, emb_c = _sc_gather(q_flat, c_rows, que_table, concept_emb)
    emb_qc, xemb, emb_qca = _tc_merge(emb_q, emb_c, r1, W, b.reshape(1, EMB))
    return (xemb.reshape(B, L, EMB),
            emb_qca.reshape(B, L, 4 * EMB),
            emb_qc.reshape(B, L, 2 * EMB),
            emb_q.reshape(B, L, EMB),
            emb_c.reshape(B, L, EMB))
